# Initial kernel scaffold; baseline (speedup 1.0000x reference)
import jax
import jax.numpy as jnp
from jax.experimental import pallas as pl


def kernel(center_words, pos_words, neg_words, v_embed, u_embed):
    raise NotImplementedError("write your pallas kernel here")


# speedup vs baseline: 3.9936x; 3.9936x over previous
import functools

import jax
import jax.numpy as jnp
from jax import lax
from jax.experimental import pallas as pl
from jax.experimental.pallas import tpu as pltpu
from jax.experimental.pallas import tpu_sc as plsc

V = 1000000
D = 64
B = 16384
K = 20

NC = 2
NS = 16
NW = NC * NS
BW = B // NW
CB = 32
NCHUNK = BW // CB
NEG_ROWS = CB * K

_mesh = plsc.VectorSubcoreMesh(
    core_axis_name="c", subcore_axis_name="s", num_cores=NC, num_subcores=NS
)


@functools.partial(
    pl.kernel,
    out_type=(
        jax.ShapeDtypeStruct((B,), jnp.float32),
        jax.ShapeDtypeStruct((B, K), jnp.float32),
    ),
    mesh=_mesh,
    compiler_params=pltpu.CompilerParams(
        needs_layout_passes=False, use_tc_tiling_on_sc=False),
    scratch_types=[
        pltpu.VMEM((NCHUNK, CB), jnp.int32),
        pltpu.VMEM((NCHUNK, CB), jnp.int32),
        pltpu.VMEM((BW * K // 128, 128), jnp.int32),
        pltpu.VMEM((CB, D), jnp.float32),
        pltpu.VMEM((CB, D), jnp.float32),
        pltpu.VMEM((NEG_ROWS, D), jnp.float32),
        pltpu.VMEM((BW,), jnp.float32),
        pltpu.VMEM((BW, K), jnp.float32),
        pltpu.SemaphoreType.DMA,
    ],
)
def _sc_scores(v_hbm, u_hbm, cidx_hbm, pidx_hbm, nidx_hbm,
               pos_out, neg_out,
               cidx, pidx, nidx, v_buf, p_buf, n_buf, pos_sc, neg_sc, sem):
    wid = lax.axis_index("s") * NC + lax.axis_index("c")

    pltpu.sync_copy(cidx_hbm.at[pl.ds(wid * NCHUNK, NCHUNK)], cidx)
    pltpu.sync_copy(pidx_hbm.at[pl.ds(wid * NCHUNK, NCHUNK)], pidx)
    nrows = BW * K // 128
    pltpu.sync_copy(nidx_hbm.at[pl.ds(wid * nrows, nrows)], nidx)

    iota = lax.iota(jnp.int32, 16)

    def chunk_body(c, carry):
        cps = [
            pltpu.async_copy(v_hbm.at[cidx.at[c]], v_buf, sem),
            pltpu.async_copy(u_hbm.at[pidx.at[c]], p_buf, sem),
        ]
        for j in range(5):
            cps.append(pltpu.async_copy(
                u_hbm.at[nidx.at[c * 5 + j]],
                n_buf.at[pl.ds(j * 128, 128)], sem))
        for cp in cps:
            cp.wait()

        for gg in range(CB // 16):
            bvec = gg * 16 + iota
            nrow0 = bvec * K
            acc0 = (jnp.zeros((16,), jnp.float32),) * (K + 1)

            def dbody(d, accs):
                dvec = jnp.full((16,), d, jnp.int32)
                vv = plsc.load_gather(v_buf, [bvec, dvec])
                pv = plsc.load_gather(p_buf, [bvec, dvec])
                out = [accs[0] + vv * pv]
                for k in range(K):
                    nv = plsc.load_gather(n_buf, [nrow0 + k, dvec])
                    out.append(accs[k + 1] + vv * nv)
                return tuple(out)

            accs = lax.fori_loop(0, D, dbody, acc0)
            base = c * CB + gg * 16
            pos_sc[pl.ds(base, 16)] = accs[0]
            blvec = base + iota
            for k in range(K):
                plsc.store_scatter(
                    neg_sc, [blvec, jnp.full((16,), k, jnp.int32)],
                    accs[k + 1])
        return carry

    lax.fori_loop(0, NCHUNK, chunk_body, 0)

    pltpu.sync_copy(pos_sc, pos_out.at[pl.ds(wid * BW, BW)])
    pltpu.sync_copy(neg_sc, neg_out.at[pl.ds(wid * BW, BW)])


def _loss_body(p_ref, n_ref, o_ref):
    def logsig(x):
        return jnp.minimum(x, 0.0) - jnp.log1p(jnp.exp(-jnp.abs(x)))

    tot = jnp.sum(logsig(p_ref[...])) + jnp.sum(logsig(-n_ref[...]))
    o_ref[0, 0] = -tot / jnp.float32(B)


_loss_call = pl.pallas_call(
    _loss_body,
    out_shape=jax.ShapeDtypeStruct((1, 1), jnp.float32),
    out_specs=pl.BlockSpec(memory_space=pltpu.MemorySpace.SMEM),
)


def kernel(center_words, pos_words, neg_words, v_embed, u_embed):
    c2 = center_words.reshape(NW * NCHUNK, CB)
    p2 = pos_words.reshape(NW * NCHUNK, CB)
    n2 = neg_words.reshape(B * K // 128, 128)
    pos_s, neg_s = _sc_scores(v_embed, u_embed, c2, p2, n2)
    loss = _loss_call(pos_s.reshape(128, 128), neg_s.reshape(B * K // 128, 128))
    return loss[0, 0]


# speedup vs baseline: 4.0821x; 1.0222x over previous
import functools

import jax
import jax.numpy as jnp
from jax import lax
from jax.experimental import pallas as pl
from jax.experimental.pallas import tpu as pltpu
from jax.experimental.pallas import tpu_sc as plsc

V = 1000000
D = 64
B = 16384
K = 20

NC = 2
NS = 16
NW = NC * NS
BW = B // NW
CB = 32
NCHUNK = BW // CB
NEG_ROWS = CB * K

_mesh = plsc.VectorSubcoreMesh(
    core_axis_name="c", subcore_axis_name="s", num_cores=NC, num_subcores=NS
)


@functools.partial(
    pl.kernel,
    out_type=(
        jax.ShapeDtypeStruct((B,), jnp.float32),
        jax.ShapeDtypeStruct((B, K), jnp.float32),
    ),
    mesh=_mesh,
    compiler_params=pltpu.CompilerParams(
        needs_layout_passes=False, use_tc_tiling_on_sc=False),
    scratch_types=[
        pltpu.VMEM((NCHUNK, CB), jnp.int32),
        pltpu.VMEM((NCHUNK, CB), jnp.int32),
        pltpu.VMEM((BW * K // 128, 128), jnp.int32),
        pltpu.VMEM((2, CB, D), jnp.float32),
        pltpu.VMEM((2, CB, D), jnp.float32),
        pltpu.VMEM((2, NEG_ROWS, D), jnp.float32),
        pltpu.VMEM((BW,), jnp.float32),
        pltpu.VMEM((BW, K), jnp.float32),
        pltpu.SemaphoreType.DMA,
        pltpu.SemaphoreType.DMA,
    ],
)
def _sc_scores(v_hbm, u_hbm, cidx_hbm, pidx_hbm, nidx_hbm,
               pos_out, neg_out,
               cidx, pidx, nidx, v_buf, p_buf, n_buf, pos_sc, neg_sc,
               sem0, sem1):
    wid = lax.axis_index("s") * NC + lax.axis_index("c")

    pltpu.sync_copy(cidx_hbm.at[pl.ds(wid * NCHUNK, NCHUNK)], cidx)
    pltpu.sync_copy(pidx_hbm.at[pl.ds(wid * NCHUNK, NCHUNK)], pidx)
    nrows = BW * K // 128
    pltpu.sync_copy(nidx_hbm.at[pl.ds(wid * nrows, nrows)], nidx)

    iota = lax.iota(jnp.int32, 16)
    sems = (sem0, sem1)

    def copies(c, s):
        sem = sems[s]
        cps = [
            pltpu.make_async_copy(v_hbm.at[cidx.at[c]], v_buf.at[s], sem),
            pltpu.make_async_copy(u_hbm.at[pidx.at[c]], p_buf.at[s], sem),
        ]
        for j in range(5):
            cps.append(pltpu.make_async_copy(
                u_hbm.at[nidx.at[c * 5 + j]],
                n_buf.at[s].at[pl.ds(j * 128, 128)], sem))
        return cps

    def issue(c, s):
        for cp in copies(c, s):
            cp.start()

    def drain(c, s):
        for cp in copies(c, s):
            cp.wait()

    def compute(c, s):
        vb, pb, nb = v_buf.at[s], p_buf.at[s], n_buf.at[s]
        for gg in range(CB // 16):
            bvec = gg * 16 + iota
            nrow0 = bvec * K
            acc0 = (jnp.zeros((16,), jnp.float32),) * (K + 1)

            def dbody(d, accs):
                dvec = jnp.full((16,), d, jnp.int32)
                vv = plsc.load_gather(vb, [bvec, dvec])
                pv = plsc.load_gather(pb, [bvec, dvec])
                out = [accs[0] + vv * pv]
                for k in range(K):
                    nv = plsc.load_gather(nb, [nrow0 + k, dvec])
                    out.append(accs[k + 1] + vv * nv)
                return tuple(out)

            accs = lax.fori_loop(0, D, dbody, acc0)
            base = c * CB + gg * 16
            pos_sc[pl.ds(base, 16)] = accs[0]
            blvec = base + iota
            for k in range(K):
                plsc.store_scatter(
                    neg_sc, [blvec, jnp.full((16,), k, jnp.int32)],
                    accs[k + 1])

    issue(0, 0)
    issue(1, 1)

    def outer(i, carry):
        cc = i * 2
        for s in range(2):
            c = cc + s
            drain(c, s)
            compute(c, s)

            @pl.when(c + 2 < NCHUNK)
            def _():
                issue(c + 2, s)
        return carry

    lax.fori_loop(0, NCHUNK // 2, outer, 0)

    pltpu.sync_copy(pos_sc, pos_out.at[pl.ds(wid * BW, BW)])
    pltpu.sync_copy(neg_sc, neg_out.at[pl.ds(wid * BW, BW)])


def _loss_body(p_ref, n_ref, o_ref):
    def logsig(x):
        return jnp.minimum(x, 0.0) - jnp.log1p(jnp.exp(-jnp.abs(x)))

    tot = jnp.sum(logsig(p_ref[...])) + jnp.sum(logsig(-n_ref[...]))
    o_ref[0, 0] = -tot / jnp.float32(B)


_loss_call = pl.pallas_call(
    _loss_body,
    out_shape=jax.ShapeDtypeStruct((1, 1), jnp.float32),
    out_specs=pl.BlockSpec(memory_space=pltpu.MemorySpace.SMEM),
)


def kernel(center_words, pos_words, neg_words, v_embed, u_embed):
    c2 = center_words.reshape(NW * NCHUNK, CB)
    p2 = pos_words.reshape(NW * NCHUNK, CB)
    n2 = neg_words.reshape(B * K // 128, 128)
    pos_s, neg_s = _sc_scores(v_embed, u_embed, c2, p2, n2)
    loss = _loss_call(pos_s.reshape(128, 128), neg_s.reshape(B * K // 128, 128))
    return loss[0, 0]
